# Initial kernel scaffold; baseline (speedup 1.0000x reference)
import jax
import jax.numpy as jnp
from jax.experimental import pallas as pl


def kernel(predictions, ref_imgs):
    raise NotImplementedError("write your pallas kernel here")


# speedup vs baseline: 5.8720x; 5.8720x over previous
import functools

import jax
import jax.numpy as jnp
from jax.experimental import pallas as pl
from jax.experimental.pallas import tpu as pltpu

_BS = 8
_L = 64
_IMG = 224
_K = 16
_BIG_I = 2 ** 30


def _ccloss_kernel(pred_ref, img_ref, out_ref):
    b = pl.program_id(0)
    l = pl.program_id(1)

    n = l * _BS + b
    b2 = n // _L
    l2 = n - b2 * _L
    px = pred_ref[b2, l2, 0]
    py = pred_ref[b2, l2, 1]
    gx = 2.0 * px - 1.0
    gy = 2.0 * py - 1.0
    fx = ((gx + 1.0) * _IMG - 1.0) / 2.0
    fy = ((gy + 1.0) * _IMG - 1.0) / 2.0
    def _round_i32(v):
        r = jnp.round(jnp.full((1, 128), v, jnp.float32)).astype(jnp.int32)
        return jnp.max(r)

    ixn = _round_i32(fx)
    iyn = _round_i32(fy)
    valid = ((ixn >= 0) & (ixn < _IMG) & (iyn >= 0) & (iyn < _IMG))
    vf = valid.astype(jnp.float32)
    ixc = jnp.clip(ixn, 0, _IMG - 1)
    iyc = jnp.clip(iyn, 0, _IMG - 1)

    lane = jax.lax.broadcasted_iota(jnp.int32, (1, _IMG), 1)
    sel = (lane == ixc).astype(jnp.float32)
    pooled = []
    for c in range(3):
        row = img_ref[0, c, pl.ds(iyc, 1), :]
        pooled.append(jnp.sum(row * sel) * vf)

    d = (img_ref[0, 0] - pooled[0]) ** 2
    d = d + (img_ref[0, 1] - pooled[1]) ** 2
    d = d + (img_ref[0, 2] - pooled[2]) ** 2

    rows = jax.lax.broadcasted_iota(jnp.int32, (_IMG, _IMG), 0)
    cols = jax.lax.broadcasted_iota(jnp.int32, (_IMG, _IMG), 1)
    lin = rows * _IMG + cols

    kiota = jax.lax.broadcasted_iota(jnp.int32, (1, _K), 1)
    xs = jnp.zeros((1, _K), jnp.float32)
    ys = jnp.zeros((1, _K), jnp.float32)
    work = d
    for k in range(_K):
        m = jnp.min(work)
        pmin = jnp.min(jnp.where(work == m, lin, _BIG_I))
        work = jnp.where(lin == pmin, jnp.float32(jnp.inf), work)
        prow = pmin // _IMG
        pcol = pmin - prow * _IMG
        xk = pcol.astype(jnp.float32) / _IMG
        yk = prow.astype(jnp.float32) / _IMG
        hit = kiota == k
        xs = jnp.where(hit, xk, xs)
        ys = jnp.where(hit, yk, ys)

    @pl.when(l < _L - 1)
    def _():
        qx = pred_ref[b, l + 1, 0]
        qy = pred_ref[b, l + 1, 1]
        dk = (qx - xs) ** 2 + (qy - ys) ** 2
        dbest = jnp.min(dk)
        kbest = jnp.min(jnp.where(dk == dbest, kiota, _BIG_I))
        hitk = kiota == kbest
        bx = jnp.sum(jnp.where(hitk, xs, 0.0))
        by = jnp.sum(jnp.where(hitk, ys, 0.0))
        out_ref[b, l + 1] = (qx - bx) ** 2 + (qy - by) ** 2

    @pl.when(l == _L - 1)
    def _():
        out_ref[b, 0] = 0.0


@jax.jit
def kernel(predictions, ref_imgs):
    contrib = pl.pallas_call(
        _ccloss_kernel,
        grid=(_BS, _L),
        in_specs=[
            pl.BlockSpec(memory_space=pltpu.SMEM),
            pl.BlockSpec((1, 3, _IMG, _IMG), lambda b, l: (b, 0, 0, 0)),
        ],
        out_specs=pl.BlockSpec(memory_space=pltpu.SMEM),
        out_shape=jax.ShapeDtypeStruct((_BS, _L), jnp.float32),
    )(predictions, ref_imgs)
    return jnp.mean(contrib[:, 1:])
